# Initial kernel scaffold; baseline (speedup 1.0000x reference)
import jax
import jax.numpy as jnp
from jax.experimental import pallas as pl


def kernel(features, edge_index, W1s, W1n, b1, W2s, W2n, b2, W3s, W3n, b3):
    raise NotImplementedError("write your pallas kernel here")


# speedup vs baseline: 3.2701x; 3.2701x over previous
import functools

import jax
import jax.numpy as jnp
from jax import lax
from jax.experimental import pallas as pl
from jax.experimental.pallas import tpu as pltpu
from jax.experimental.pallas import tpu_sc as plsc

N_NODES = 10000
D = 128
E = 320000

NC = 2
NS = 16
NW = NC * NS
CHUNK = 128
ROWS_PER_TILE = 640
NP = NS * ROWS_PER_TILE
EPW = ((E // NW + CHUNK - 1) // CHUNK) * CHUNK
EP = EPW * NW
NCHUNK = EPW // CHUNK

_MESH = plsc.VectorSubcoreMesh(core_axis_name="c", subcore_axis_name="s",
                               num_cores=NC, num_subcores=NS)


def _sc_agg_body(x_hbm, src_hbm, dst_hbm, zrows_hbm, agg_hbm,
                 sidx_v, didx_v, rows_v, acc_sh, sem):
    c = lax.axis_index("c")
    s = lax.axis_index("s")
    wid = c * NS + s
    row0 = s * ROWS_PER_TILE

    pltpu.sync_copy(zrows_hbm, rows_v)
    for k in range(ROWS_PER_TILE // CHUNK):
        pltpu.sync_copy(rows_v, acc_sh.at[pl.ds(row0 + k * CHUNK, CHUNK)])
    plsc.subcore_barrier()

    ebase = wid * EPW

    def step(j, carry):
        off = ebase + j * CHUNK
        pltpu.sync_copy(src_hbm.at[pl.ds(off, CHUNK)], sidx_v)
        pltpu.sync_copy(dst_hbm.at[pl.ds(off, CHUNK)], didx_v)
        pltpu.async_copy(x_hbm.at[sidx_v], rows_v, sem).wait()
        pltpu.sync_copy(rows_v, acc_sh.at[didx_v], add=True)
        return carry

    lax.fori_loop(0, NCHUNK, step, 0)
    plsc.subcore_barrier()

    for k in range(ROWS_PER_TILE // CHUNK):
        r = row0 + k * CHUNK
        pltpu.sync_copy(acc_sh.at[pl.ds(r, CHUNK)], rows_v)
        pltpu.sync_copy(rows_v, agg_hbm.at[c, pl.ds(r, CHUNK)])


_sc_agg = pl.kernel(
    _sc_agg_body,
    out_type=jax.ShapeDtypeStruct((NC, NP, D), jnp.float32),
    mesh=_MESH,
    scratch_types=[
        pltpu.VMEM((CHUNK,), jnp.int32),
        pltpu.VMEM((CHUNK,), jnp.int32),
        pltpu.VMEM((CHUNK, D), jnp.float32),
        pltpu.VMEM_SHARED((NP, D), jnp.float32),
        pltpu.SemaphoreType.DMA,
    ],
)


def _tc_dense_body(relu, x_ref, a0_ref, a1_ref, d0_ref, d1_ref,
                   ws_ref, wn_ref, b_ref, o_ref):
    a = a0_ref[0] + a1_ref[0]
    dsum = d0_ref[0, :, 0:1] + d1_ref[0, :, 0:1]
    mean = a * (1.0 / jnp.maximum(dsum, 1.0))
    out = (jnp.dot(x_ref[...], ws_ref[...], preferred_element_type=jnp.float32)
           + jnp.dot(mean, wn_ref[...], preferred_element_type=jnp.float32)
           + b_ref[...])
    if relu:
        out = jnp.maximum(out, 0.0)
    o_ref[...] = out


def _tc_dense(x, agg, deg, ws, wn, b, relu):
    BR = 1280
    grid = (NP // BR,)
    return pl.pallas_call(
        functools.partial(_tc_dense_body, relu),
        grid=grid,
        in_specs=[
            pl.BlockSpec((BR, D), lambda i: (i, 0)),
            pl.BlockSpec((1, BR, D), lambda i: (0, i, 0)),
            pl.BlockSpec((1, BR, D), lambda i: (1, i, 0)),
            pl.BlockSpec((1, BR, D), lambda i: (0, i, 0)),
            pl.BlockSpec((1, BR, D), lambda i: (1, i, 0)),
            pl.BlockSpec((D, D), lambda i: (0, 0)),
            pl.BlockSpec((D, D), lambda i: (0, 0)),
            pl.BlockSpec((1, D), lambda i: (0, 0)),
        ],
        out_specs=pl.BlockSpec((BR, D), lambda i: (i, 0)),
        out_shape=jax.ShapeDtypeStruct((NP, D), jnp.float32),
    )(x, agg, agg, deg, deg, ws, wn, b)


def kernel(features, edge_index, W1s, W1n, b1, W2s, W2n, b2, W3s, W3n, b3):
    src = edge_index[0].astype(jnp.int32)
    dst = edge_index[1].astype(jnp.int32)
    src = jnp.pad(src, (0, EP - E))
    dst = jnp.pad(dst, (0, EP - E), constant_values=N_NODES)
    x0 = jnp.pad(features, ((0, NP - N_NODES), (0, 0)))

    zrows = jnp.zeros((CHUNK, D), jnp.float32)
    ones_tab = jnp.ones((NP, D), jnp.float32)

    deg = _sc_agg(ones_tab, src, dst, zrows)
    agg1 = _sc_agg(x0, src, dst, zrows)
    x1 = _tc_dense(x0, agg1, deg, W1s, W1n, b1.reshape(1, D), relu=True)
    agg2 = _sc_agg(x1, src, dst, zrows)
    x2 = _tc_dense(x1, agg2, deg, W2s, W2n, b2.reshape(1, D), relu=True)
    agg3 = _sc_agg(x2, src, dst, zrows)
    x3 = _tc_dense(x2, agg3, deg, W3s, W3n, b3.reshape(1, D), relu=False)
    return x3[:N_NODES]
